# baseline (speedup 1.0000x reference)
import functools

import jax
import jax.numpy as jnp
from jax import lax
from jax.experimental import pallas as pl
from jax.experimental.pallas import tpu as pltpu
from jax.experimental.pallas import tpu_sc as plsc

_D = 128
_IW = 16
_CHUNK = 128
_NW = 32
_NB = 4
_L = 16


def _gather_body(nchunks, tail, x_hbm, inds_hbm, out_hbm, *refs):
  offs = refs[0:_NB]
  idxv = refs[_NB:2 * _NB]
  rows = refs[2 * _NB:3 * _NB]
  isem = refs[3 * _NB:4 * _NB]
  gsem = refs[4 * _NB:5 * _NB]
  ssem = refs[5 * _NB:6 * _NB]

  big = (nchunks + _NW - 1) // _NW
  cut = nchunks - (big - 1) * _NW

  w = lax.axis_index("s") * 2 + lax.axis_index("c")
  nj = jnp.where(w < cut, big, big - 1)
  base_chunk = jnp.where(w < cut, w * big, cut * big + (w - cut) * (big - 1))

  def start_idx(j, p):
    gc = base_chunk + j
    for g in range(_CHUNK // _L):
      offs[p][pl.ds(g * _L, _L)] = (
          lax.iota(jnp.int32, _L) * _IW + (gc * _CHUNK + g * _L) * _IW)
    if tail == _CHUNK:
      pltpu.async_copy(inds_hbm.at[offs[p]], idxv[p], isem[p])
    else:
      @pl.when(gc != nchunks - 1)
      def _f():
        pltpu.async_copy(inds_hbm.at[offs[p]], idxv[p], isem[p])

      @pl.when(gc == nchunks - 1)
      def _t():
        pltpu.async_copy(inds_hbm.at[offs[p].at[pl.ds(0, tail)]],
                         idxv[p].at[pl.ds(0, tail)], isem[p])

  def wait_idx(j, p):
    gc = base_chunk + j
    if tail == _CHUNK:
      pltpu.make_async_copy(inds_hbm.at[offs[p]], idxv[p], isem[p]).wait()
    else:
      @pl.when(gc != nchunks - 1)
      def _f():
        pltpu.make_async_copy(inds_hbm.at[offs[p]], idxv[p], isem[p]).wait()

      @pl.when(gc == nchunks - 1)
      def _t():
        pltpu.make_async_copy(inds_hbm.at[offs[p].at[pl.ds(0, tail)]],
                              idxv[p].at[pl.ds(0, tail)], isem[p]).wait()

  def start_gather(p):
    pltpu.async_copy(x_hbm.at[idxv[p]], rows[p], gsem[p])

  def wait_gather(p):
    pltpu.make_async_copy(x_hbm.at[idxv[p]], rows[p], gsem[p]).wait()

  def start_store(j, p):
    gc = base_chunk + j
    if tail == _CHUNK:
      pltpu.async_copy(rows[p], out_hbm.at[pl.ds(gc * _CHUNK, _CHUNK)], ssem[p])
    else:
      @pl.when(gc != nchunks - 1)
      def _f():
        pltpu.async_copy(rows[p], out_hbm.at[pl.ds(gc * _CHUNK, _CHUNK)],
                         ssem[p])

      @pl.when(gc == nchunks - 1)
      def _t():
        pltpu.async_copy(rows[p].at[pl.ds(0, tail)],
                         out_hbm.at[pl.ds(gc * _CHUNK, tail)], ssem[p])

  def wait_store(p, is_tail):
    n = tail if is_tail else _CHUNK
    pltpu.make_async_copy(rows[p].at[pl.ds(0, n)],
                          out_hbm.at[pl.ds(0, n)], ssem[p]).wait()

  if tail != _CHUNK:
    @pl.when(base_chunk + nj - 1 == nchunks - 1)
    def _zfill():
      z = jnp.zeros((_L,), jnp.int32)
      for p in range(_NB):
        for g in range(_CHUNK // _L):
          idxv[p][pl.ds(g * _L, _L)] = z

  for p in range(_NB - 1):
    @pl.when(p < nj)
    def _pi(p=p):
      start_idx(p, p)
  for p in range(_NB - 2):
    @pl.when(p < nj)
    def _pg(p=p):
      wait_idx(p, p)
      start_gather(p)

  nrounds = (nj + _NB - 1) // _NB

  @pl.loop(0, nrounds)
  def _round(r):
    for p in range(_NB):
      j = r * _NB + p

      @pl.when(j < nj)
      def _body(j=j, p=p):
        pm1 = (p - 1) % _NB
        pm2 = (p - 2) % _NB

        @pl.when(j >= 1)
        def _drain_prev():
          wait_store(pm1, False)

        @pl.when(j + _NB - 1 < nj)
        def _prefetch_i():
          start_idx(j + _NB - 1, pm1)

        @pl.when(j + _NB - 2 < nj)
        def _launch_g():
          wait_idx(j + _NB - 2, pm2)
          start_gather(pm2)

        wait_gather(p)
        start_store(j, p)

  last_p = lax.rem(nj - 1, _NB)
  last_is_tail = (base_chunk + nj - 1) == (nchunks - 1)
  for p in range(_NB):
    @pl.when(last_p == p)
    def _drain_last(p=p):
      if tail == _CHUNK:
        wait_store(p, False)
      else:
        @pl.when(last_is_tail)
        def _t():
          wait_store(p, True)

        @pl.when(jnp.logical_not(last_is_tail))
        def _f():
          wait_store(p, False)


@functools.partial(jax.jit, static_argnums=(2, 3, 4))
def _gather(x, inds, n_out, nchunks, tail):
  mesh = plsc.VectorSubcoreMesh(core_axis_name="c", subcore_axis_name="s")
  run = pl.kernel(
      functools.partial(_gather_body, nchunks, tail),
      out_type=jax.ShapeDtypeStruct((n_out, _D), jnp.float32),
      mesh=mesh,
      scratch_types=[pltpu.VMEM((_CHUNK,), jnp.int32) for _ in range(2 * _NB)]
      + [pltpu.VMEM((_CHUNK, _D), jnp.float32) for _ in range(_NB)]
      + [pltpu.SemaphoreType.DMA for _ in range(3 * _NB)],
  )
  return run(x, inds)


def kernel(x, upsample_inds):
  n_out = upsample_inds.shape[0]
  inds = upsample_inds.astype(jnp.int32).reshape(-1)
  nchunks = (n_out + _CHUNK - 1) // _CHUNK
  tail = n_out - (nchunks - 1) * _CHUNK
  return _gather(x, inds, n_out, nchunks, tail)


# speedup vs baseline: 1.8313x; 1.8313x over previous
import functools

import jax
import jax.numpy as jnp
from jax import lax
from jax.experimental import pallas as pl
from jax.experimental.pallas import tpu as pltpu
from jax.experimental.pallas import tpu_sc as plsc

_D = 128
_CHUNK = 128
_NW = 32
_NB = 6


def _gather_body(nchunks, tail, x_hbm, idx_hbm, out_hbm, *refs):
  ibuf = refs[0]
  rows = refs[1:1 + _NB]
  gsem = refs[1 + _NB:1 + 2 * _NB]
  ssem = refs[1 + 2 * _NB:1 + 3 * _NB]

  big = (nchunks + _NW - 1) // _NW
  cut = nchunks - (big - 1) * _NW

  w = lax.axis_index("s") * 2 + lax.axis_index("c")
  nj = jnp.where(w < cut, big, big - 1)
  base_chunk = jnp.where(w < cut, w * big, cut * big + (w - cut) * (big - 1))

  pltpu.sync_copy(idx_hbm.at[pl.ds(base_chunk * _CHUNK, big * _CHUNK)], ibuf)

  def start_gather(j, p):
    pltpu.async_copy(
        x_hbm.at[ibuf.at[pl.ds(j * _CHUNK, _CHUNK)]], rows[p], gsem[p])

  def wait_gather(p):
    pltpu.make_async_copy(
        x_hbm.at[ibuf.at[pl.ds(0, _CHUNK)]], rows[p], gsem[p]).wait()

  def start_store(j, p):
    gc = base_chunk + j
    if tail == _CHUNK:
      pltpu.async_copy(rows[p], out_hbm.at[pl.ds(gc * _CHUNK, _CHUNK)], ssem[p])
    else:
      @pl.when(gc == nchunks - 1)
      def _t():
        pltpu.async_copy(rows[p].at[pl.ds(0, tail)],
                         out_hbm.at[pl.ds(gc * _CHUNK, tail)], ssem[p])

      @pl.when(gc != nchunks - 1)
      def _f():
        pltpu.async_copy(rows[p], out_hbm.at[pl.ds(gc * _CHUNK, _CHUNK)],
                         ssem[p])

  def wait_store(p, is_tail):
    n = tail if is_tail else _CHUNK
    pltpu.make_async_copy(rows[p].at[pl.ds(0, n)],
                          out_hbm.at[pl.ds(0, n)], ssem[p]).wait()

  for p in range(_NB - 1):
    @pl.when(p < nj)
    def _prime(p=p):
      start_gather(p, p)

  nrounds = (nj + _NB - 1) // _NB

  @pl.loop(0, nrounds)
  def _round(r):
    for p in range(_NB):
      j = r * _NB + p

      @pl.when(j < nj)
      def _body(j=j, p=p):
        pm1 = (p - 1) % _NB

        @pl.when(j >= 1)
        def _drain_prev():
          wait_store(pm1, False)

        @pl.when(j + _NB - 1 < nj)
        def _prefetch():
          start_gather(j + _NB - 1, pm1)

        wait_gather(p)
        start_store(j, p)

  last_p = lax.rem(nj - 1, _NB)
  last_is_tail = (base_chunk + nj - 1) == (nchunks - 1)
  for p in range(_NB):
    @pl.when(last_p == p)
    def _drain_last(p=p):
      if tail == _CHUNK:
        wait_store(p, False)
      else:
        @pl.when(last_is_tail)
        def _t():
          wait_store(p, True)

        @pl.when(jnp.logical_not(last_is_tail))
        def _f():
          wait_store(p, False)


@functools.partial(jax.jit, static_argnums=(2, 3, 4))
def _gather(x, idx_pad, n_out, nchunks, tail):
  big = (nchunks + _NW - 1) // _NW
  mesh = plsc.VectorSubcoreMesh(core_axis_name="c", subcore_axis_name="s")
  run = pl.kernel(
      functools.partial(_gather_body, nchunks, tail),
      out_type=jax.ShapeDtypeStruct((n_out, _D), jnp.float32),
      mesh=mesh,
      scratch_types=[pltpu.VMEM((big * _CHUNK,), jnp.int32)]
      + [pltpu.VMEM((_CHUNK, _D), jnp.float32) for _ in range(_NB)]
      + [pltpu.SemaphoreType.DMA for _ in range(2 * _NB)],
  )
  return run(x, idx_pad)


def kernel(x, upsample_inds):
  n_out = upsample_inds.shape[0]
  idx = upsample_inds[:, 0].astype(jnp.int32)
  nchunks = (n_out + _CHUNK - 1) // _CHUNK
  tail = n_out - (nchunks - 1) * _CHUNK
  big = (nchunks + _NW - 1) // _NW
  cut = nchunks - (big - 1) * _NW
  last_base = cut * big + (_NW - 1 - cut) * (big - 1)
  pad_len = (last_base + big) * _CHUNK
  idx_pad = jnp.pad(idx, (0, pad_len - n_out))
  return _gather(x, idx_pad, n_out, nchunks, tail)
